# baseline (speedup 1.0000x reference)
import functools

import jax
import jax.numpy as jnp
from jax import lax
from jax.experimental import pallas as pl
from jax.experimental.pallas import tpu as pltpu
from jax.experimental.pallas import tpu_sc as plsc

_N = 10000
_E = 320000
_D = 128
_H = 64

_NC = 2
_NS = 16
_NW = _NC * _NS
_EPT = _E // _NW
_CH = 128
_NFULL = _EPT // _CH
_REM = _EPT - _NFULL * _CH
_RPT = 624
_RTAIL = _N - _NS * _RPT

_MIN32 = -(2 ** 31)

_HROWS = _E // _D


def _hist_body(colb_ref, out_ref):
    ioc = lax.broadcasted_iota(jnp.int32, (_D, 1), 0)

    def body(r, acc):
        crow = colb_ref[pl.ds(r, 1), :]
        ohi = ((crow >> 7) == ioc).astype(jnp.bfloat16)
        olo = ((crow & 127) == ioc).astype(jnp.bfloat16)
        return acc + lax.dot_general(
            ohi, olo, (((1,), (1,)), ((), ())),
            preferred_element_type=jnp.float32)

    out_ref[...] = lax.fori_loop(0, _HROWS, body,
                                 jnp.zeros((_D, _D), jnp.float32))


def _hist_call(col2d):
    return pl.pallas_call(
        _hist_body,
        out_shape=jax.ShapeDtypeStruct((_D, _D), jnp.float32),
    )(col2d)


def _fill_idx(dst, src, off):
    for k in range(_CH // 16):
        dst[pl.ds(k * 16, 16)] = src[pl.ds(off + k * 16, 16)]


def _segsum_kernel(y_hbm, row_hbm, col_hbm, zeros_hbm, out_hbm,
                   idxc, rowv0, rowv1, colv, rowr, colr,
                   gbuf0, gbuf1, gbufr, acc_sh, sem0, sem1):
    c = lax.axis_index("c")
    s = lax.axis_index("s")
    pltpu.sync_copy(zeros_hbm, acc_sh.at[pl.ds(s * _RPT, _RPT)])

    @pl.when(s == _NS - 1)
    def _():
        pltpu.sync_copy(zeros_hbm.at[pl.ds(0, _RTAIL)],
                        acc_sh.at[pl.ds(_NS * _RPT, _RTAIL)])

    base = (c * _NS + s) * _EPT
    pltpu.sync_copy(col_hbm.at[pl.ds(base, _NFULL * _CH)], idxc)
    plsc.subcore_barrier()

    bufs = ((rowv0, gbuf0, sem0), (rowv1, gbuf1, sem1))

    def _issue(j, which):
        rowv, gbuf, sem = bufs[which]
        pltpu.sync_copy(row_hbm.at[pl.ds(base + j * _CH, _CH)], rowv)
        pltpu.async_copy(y_hbm.at[rowv], gbuf, sem)

    def _drain(j, which, last):
        rowv, gbuf, sem = bufs[which]
        pltpu.make_async_copy(y_hbm.at[rowv], gbuf, sem).wait()
        _fill_idx(colv, idxc, j * _CH)
        pltpu.sync_copy(gbuf, acc_sh.at[colv], add=True)

        @pl.when(jnp.logical_not(last))
        def _():
            _issue(j + 2, which)

    _issue(0, 0)
    _issue(1, 1)

    def body(t, _):
        j = 2 * t
        _drain(j, 0, j + 2 >= _NFULL)
        _drain(j + 1, 1, j + 3 >= _NFULL)
        return 0

    lax.fori_loop(0, _NFULL // 2, body, 0)
    b = base + _NFULL * _CH
    pltpu.sync_copy(row_hbm.at[pl.ds(b, _REM)], rowr)
    pltpu.sync_copy(col_hbm.at[pl.ds(b, _REM)], colr)
    pltpu.async_copy(y_hbm.at[rowr], gbufr, sem0).wait()
    pltpu.sync_copy(gbufr, acc_sh.at[colr], add=True)
    plsc.subcore_barrier()
    pltpu.sync_copy(acc_sh.at[pl.ds(s * _RPT, _RPT)],
                    out_hbm.at[c, pl.ds(s * _RPT, _RPT)])

    @pl.when(s == _NS - 1)
    def _():
        pltpu.sync_copy(acc_sh.at[pl.ds(_NS * _RPT, _RTAIL)],
                        out_hbm.at[c, pl.ds(_NS * _RPT, _RTAIL)])


@functools.cache
def _sc_kernels():
    mesh = plsc.VectorSubcoreMesh(core_axis_name="c", subcore_axis_name="s",
                                  num_cores=_NC, num_subcores=_NS)
    segsum = pl.kernel(
        _segsum_kernel,
        out_type=jax.ShapeDtypeStruct((_NC, _N, _D), jnp.float32),
        mesh=mesh,
        scratch_types=[
            pltpu.VMEM((_NFULL * _CH,), jnp.int32),
            pltpu.VMEM((_CH,), jnp.int32),
            pltpu.VMEM((_CH,), jnp.int32),
            pltpu.VMEM((_CH,), jnp.int32),
            pltpu.VMEM((_REM,), jnp.int32),
            pltpu.VMEM((_REM,), jnp.int32),
            pltpu.VMEM((_CH, _D), jnp.float32),
            pltpu.VMEM((_CH, _D), jnp.float32),
            pltpu.VMEM((_REM, _D), jnp.float32),
            pltpu.VMEM_SHARED((_N, _D), jnp.float32),
            pltpu.SemaphoreType.DMA,
            pltpu.SemaphoreType.DMA,
        ],
    )
    return segsum


_MM_BLK = 2000


def _y_body(x_ref, w_ref, deg1_ref, y_ref):
    xw = jnp.dot(x_ref[...], w_ref[...], preferred_element_type=jnp.float32)
    deg = deg1_ref[...] + 1.0
    y_ref[...] = xw * lax.rsqrt(deg)


def _y_call(x, w, deg1):
    grid = (_N // _MM_BLK,)
    return pl.pallas_call(
        _y_body,
        grid=grid,
        in_specs=[
            pl.BlockSpec((_MM_BLK, _D), lambda i: (i, 0)),
            pl.BlockSpec((_D, _D), lambda i: (0, 0)),
            pl.BlockSpec((_MM_BLK, 1), lambda i: (i, 0)),
        ],
        out_specs=pl.BlockSpec((_MM_BLK, _D), lambda i: (i, 0)),
        out_shape=jax.ShapeDtypeStruct((_N, _D), jnp.float32),
    )(x, w, deg1)


def _f2u(b):
    return jnp.where(b >= 0, b ^ _MIN32, ~b)


def _u2f(u):
    b = jnp.where(u < 0, u ^ _MIN32, ~u)
    return lax.bitcast_convert_type(b, jnp.float32)


def _final_body(accp_ref, y_ref, deg1_ref, bg_ref, w1_ref, b1_ref,
                w2_ref, b2_ref, w3t_ref, b3_ref, out_ref, u_ref):
    deg = deg1_ref[...] + 1.0
    dis = lax.rsqrt(deg)
    conv = (accp_ref[0] + accp_ref[1] + y_ref[...]) * dis + bg_ref[...]
    b = lax.bitcast_convert_type(conv, jnp.int32)
    u_ref[...] = _f2u(b)

    khalf = (_N // 2) - 1

    def bit_body(i, carry):
        prefix, kk, mh = carry
        bit = lax.shift_left(jnp.int32(1), jnp.int32(31) - i)
        u = u_ref[...]
        match = (u & mh) == prefix
        is0 = (u & bit) == 0
        cnt0 = jnp.sum(jnp.where(match & is0, 1.0, 0.0), axis=0,
                       keepdims=True)
        go1 = kk >= cnt0
        prefix = jnp.where(go1, prefix | bit, prefix)
        kk = jnp.where(go1, kk - cnt0, kk)
        return prefix, kk, mh | bit

    prefix0 = jnp.zeros((1, _D), jnp.int32)
    kk0 = jnp.full((1, _D), float(khalf), jnp.float32)
    key, kkf, _ = lax.fori_loop(0, 32, bit_body,
                                (prefix0, kk0, jnp.int32(0)))

    u = u_ref[...]
    v1 = _u2f(key)
    c_eq = jnp.sum(jnp.where(u == key, 1.0, 0.0), axis=0, keepdims=True)
    below = float(khalf) - kkf
    has2 = (below + c_eq) >= float(khalf + 2)

    us = u ^ _MIN32
    keys_s = key ^ _MIN32
    cand = jnp.where(us > keys_s, us, jnp.int32(2 ** 31 - 1))
    v2 = _u2f(jnp.min(cand, axis=0, keepdims=True) ^ _MIN32)
    v2 = jnp.where(has2, v1, v2)
    med = 0.5 * (v1 + v2)

    h1 = jnp.tanh(jnp.dot(med, w1_ref[...],
                          preferred_element_type=jnp.float32) + b1_ref[...])
    h2 = jnp.tanh(jnp.dot(h1, w2_ref[...],
                          preferred_element_type=jnp.float32) + b2_ref[...])
    out_ref[...] = (jnp.sum(h2 * w3t_ref[...], axis=1, keepdims=True)
                    + b3_ref[...])


def _final_call(accp, y, deg1, bg, w1, b1, w2, b2, w3t, b3):
    return pl.pallas_call(
        _final_body,
        out_shape=jax.ShapeDtypeStruct((1, 1), jnp.float32),
        scratch_shapes=[pltpu.VMEM((_N, _D), jnp.int32)],
    )(accp, y, deg1, bg, w1, b1, w2, b2, w3t, b3)


def kernel(x, edge_index, W_gcn, b_gcn, W1, b1, W2, b2, W3, b3):
    row = edge_index[0]
    col = edge_index[1]
    zerosD = jnp.zeros((_RPT, _D), jnp.float32)

    segsum_k = _sc_kernels()
    hist = _hist_call(col.reshape(_E // _D, _D))
    deg1 = hist.reshape(-1)[:_N].reshape(_N, 1)
    y = _y_call(x, W_gcn, deg1)
    accp = segsum_k(y, row, col, zerosD)
    out = _final_call(
        accp, y, deg1,
        b_gcn.reshape(1, _D),
        W1, b1.reshape(1, _H),
        W2, b2.reshape(1, _H),
        W3.reshape(1, _H), b3.reshape(1, 1),
    )
    return out


# speedup vs baseline: 1.9652x; 1.9652x over previous
import functools

import jax
import jax.numpy as jnp
from jax import lax
from jax.experimental import pallas as pl
from jax.experimental.pallas import tpu as pltpu
from jax.experimental.pallas import tpu_sc as plsc

_N = 10000
_E = 320000
_D = 128
_H = 64

_NC = 2
_NS = 16
_NW = _NC * _NS
_EPT = _E // _NW
_CH = 128
_NFULL = _EPT // _CH
_REM = _EPT - _NFULL * _CH
_RPT = 624
_RTAIL = _N - _NS * _RPT

_MIN32 = -(2 ** 31)

_HROWS = _E // _D


_HUNROLL = 10


def _hist_body(colb_ref, out_ref):
    ioc = lax.broadcasted_iota(jnp.int32, (_D, 1), 0)

    def body(t, accs):
        a0, a1 = accs
        base = t * _HUNROLL
        for k in range(_HUNROLL):
            crow = colb_ref[pl.ds(base + k, 1), :]
            ohi = ((crow >> 7) == ioc).astype(jnp.bfloat16)
            olo = ((crow & 127) == ioc).astype(jnp.bfloat16)
            h = lax.dot_general(ohi, olo, (((1,), (1,)), ((), ())),
                                preferred_element_type=jnp.float32)
            if k % 2 == 0:
                a0 = a0 + h
            else:
                a1 = a1 + h
        return a0, a1

    z = jnp.zeros((_D, _D), jnp.float32)
    a0, a1 = lax.fori_loop(0, _HROWS // _HUNROLL, body, (z, z))
    out_ref[...] = a0 + a1


def _hist_call(col2d):
    return pl.pallas_call(
        _hist_body,
        out_shape=jax.ShapeDtypeStruct((_D, _D), jnp.float32),
    )(col2d)


def _fill_idx(dst, src, off):
    for k in range(_CH // 16):
        dst[pl.ds(k * 16, 16)] = src[pl.ds(off + k * 16, 16)]


def _segsum_kernel(y_hbm, row_hbm, col_hbm, zeros_hbm, out_hbm,
                   idxc, rowv0, rowv1, colv, rowr, colr,
                   gbuf0, gbuf1, gbufr, acc_sh, sem0, sem1):
    c = lax.axis_index("c")
    s = lax.axis_index("s")
    pltpu.sync_copy(zeros_hbm, acc_sh.at[pl.ds(s * _RPT, _RPT)])

    @pl.when(s == _NS - 1)
    def _():
        pltpu.sync_copy(zeros_hbm.at[pl.ds(0, _RTAIL)],
                        acc_sh.at[pl.ds(_NS * _RPT, _RTAIL)])

    base = (c * _NS + s) * _EPT
    pltpu.sync_copy(col_hbm.at[pl.ds(base, _NFULL * _CH)], idxc)
    plsc.subcore_barrier()

    bufs = ((rowv0, gbuf0, sem0), (rowv1, gbuf1, sem1))

    def _issue(j, which):
        rowv, gbuf, sem = bufs[which]
        pltpu.sync_copy(row_hbm.at[pl.ds(base + j * _CH, _CH)], rowv)
        pltpu.async_copy(y_hbm.at[rowv], gbuf, sem)

    def _drain(j, which, last):
        rowv, gbuf, sem = bufs[which]
        pltpu.make_async_copy(y_hbm.at[rowv], gbuf, sem).wait()
        _fill_idx(colv, idxc, j * _CH)
        pltpu.sync_copy(gbuf, acc_sh.at[colv], add=True)

        @pl.when(jnp.logical_not(last))
        def _():
            _issue(j + 2, which)

    _issue(0, 0)
    _issue(1, 1)

    def body(t, _):
        j = 2 * t
        _drain(j, 0, j + 2 >= _NFULL)
        _drain(j + 1, 1, j + 3 >= _NFULL)
        return 0

    lax.fori_loop(0, _NFULL // 2, body, 0)
    b = base + _NFULL * _CH
    pltpu.sync_copy(row_hbm.at[pl.ds(b, _REM)], rowr)
    pltpu.sync_copy(col_hbm.at[pl.ds(b, _REM)], colr)
    pltpu.async_copy(y_hbm.at[rowr], gbufr, sem0).wait()
    pltpu.sync_copy(gbufr, acc_sh.at[colr], add=True)
    plsc.subcore_barrier()
    pltpu.sync_copy(acc_sh.at[pl.ds(s * _RPT, _RPT)],
                    out_hbm.at[c, pl.ds(s * _RPT, _RPT)])

    @pl.when(s == _NS - 1)
    def _():
        pltpu.sync_copy(acc_sh.at[pl.ds(_NS * _RPT, _RTAIL)],
                        out_hbm.at[c, pl.ds(_NS * _RPT, _RTAIL)])


@functools.cache
def _sc_kernels():
    mesh = plsc.VectorSubcoreMesh(core_axis_name="c", subcore_axis_name="s",
                                  num_cores=_NC, num_subcores=_NS)
    segsum = pl.kernel(
        _segsum_kernel,
        out_type=jax.ShapeDtypeStruct((_NC, _N, _D), jnp.float32),
        mesh=mesh,
        scratch_types=[
            pltpu.VMEM((_NFULL * _CH,), jnp.int32),
            pltpu.VMEM((_CH,), jnp.int32),
            pltpu.VMEM((_CH,), jnp.int32),
            pltpu.VMEM((_CH,), jnp.int32),
            pltpu.VMEM((_REM,), jnp.int32),
            pltpu.VMEM((_REM,), jnp.int32),
            pltpu.VMEM((_CH, _D), jnp.float32),
            pltpu.VMEM((_CH, _D), jnp.float32),
            pltpu.VMEM((_REM, _D), jnp.float32),
            pltpu.VMEM_SHARED((_N, _D), jnp.float32),
            pltpu.SemaphoreType.DMA,
            pltpu.SemaphoreType.DMA,
        ],
    )
    return segsum


_MM_BLK = 2000


def _y_body(x_ref, w_ref, deg1_ref, y_ref):
    xw = jnp.dot(x_ref[...], w_ref[...], preferred_element_type=jnp.float32)
    deg = deg1_ref[...] + 1.0
    y_ref[...] = xw * lax.rsqrt(deg)


def _y_call(x, w, deg1):
    grid = (_N // _MM_BLK,)
    return pl.pallas_call(
        _y_body,
        grid=grid,
        in_specs=[
            pl.BlockSpec((_MM_BLK, _D), lambda i: (i, 0)),
            pl.BlockSpec((_D, _D), lambda i: (0, 0)),
            pl.BlockSpec((_MM_BLK, 1), lambda i: (i, 0)),
        ],
        out_specs=pl.BlockSpec((_MM_BLK, _D), lambda i: (i, 0)),
        out_shape=jax.ShapeDtypeStruct((_N, _D), jnp.float32),
    )(x, w, deg1)


def _f2u(b):
    return jnp.where(b >= 0, b ^ _MIN32, ~b)


def _u2f(u):
    b = jnp.where(u < 0, u ^ _MIN32, ~u)
    return lax.bitcast_convert_type(b, jnp.float32)


def _final_body(accp_ref, y_ref, deg1_ref, bg_ref, w1_ref, b1_ref,
                w2_ref, b2_ref, w3t_ref, b3_ref, out_ref, u_ref):
    deg = deg1_ref[...] + 1.0
    dis = lax.rsqrt(deg)
    conv = (accp_ref[0] + accp_ref[1] + y_ref[...]) * dis + bg_ref[...]
    b = lax.bitcast_convert_type(conv, jnp.int32)
    u_ref[...] = _f2u(b)

    khalf = (_N // 2) - 1

    def bit_body(i, carry):
        prefix, kk, mh = carry
        bit = lax.shift_left(jnp.int32(1), jnp.int32(31) - i)
        u = u_ref[...]
        match = (u & mh) == prefix
        is0 = (u & bit) == 0
        cnt0 = jnp.sum(jnp.where(match & is0, 1.0, 0.0), axis=0,
                       keepdims=True)
        go1 = kk >= cnt0
        prefix = jnp.where(go1, prefix | bit, prefix)
        kk = jnp.where(go1, kk - cnt0, kk)
        return prefix, kk, mh | bit

    prefix0 = jnp.zeros((1, _D), jnp.int32)
    kk0 = jnp.full((1, _D), float(khalf), jnp.float32)
    key, kkf, _ = lax.fori_loop(0, 32, bit_body,
                                (prefix0, kk0, jnp.int32(0)))

    u = u_ref[...]
    v1 = _u2f(key)
    c_eq = jnp.sum(jnp.where(u == key, 1.0, 0.0), axis=0, keepdims=True)
    below = float(khalf) - kkf
    has2 = (below + c_eq) >= float(khalf + 2)

    us = u ^ _MIN32
    keys_s = key ^ _MIN32
    cand = jnp.where(us > keys_s, us, jnp.int32(2 ** 31 - 1))
    v2 = _u2f(jnp.min(cand, axis=0, keepdims=True) ^ _MIN32)
    v2 = jnp.where(has2, v1, v2)
    med = 0.5 * (v1 + v2)

    h1 = jnp.tanh(jnp.dot(med, w1_ref[...],
                          preferred_element_type=jnp.float32) + b1_ref[...])
    h2 = jnp.tanh(jnp.dot(h1, w2_ref[...],
                          preferred_element_type=jnp.float32) + b2_ref[...])
    out_ref[...] = (jnp.sum(h2 * w3t_ref[...], axis=1, keepdims=True)
                    + b3_ref[...])


def _final_call(accp, y, deg1, bg, w1, b1, w2, b2, w3t, b3):
    return pl.pallas_call(
        _final_body,
        out_shape=jax.ShapeDtypeStruct((1, 1), jnp.float32),
        scratch_shapes=[pltpu.VMEM((_N, _D), jnp.int32)],
    )(accp, y, deg1, bg, w1, b1, w2, b2, w3t, b3)


def kernel(x, edge_index, W_gcn, b_gcn, W1, b1, W2, b2, W3, b3):
    row = edge_index[0]
    col = edge_index[1]
    zerosD = jnp.zeros((_RPT, _D), jnp.float32)

    segsum_k = _sc_kernels()
    hist = _hist_call(col.reshape(_E // _D, _D))
    deg1 = hist.reshape(-1)[:_N].reshape(_N, 1)
    y = _y_call(x, W_gcn, deg1)
    accp = segsum_k(y, row, col, zerosD)
    out = _final_call(
        accp, y, deg1,
        b_gcn.reshape(1, _D),
        W1, b1.reshape(1, _H),
        W2, b2.reshape(1, _H),
        W3.reshape(1, _H), b3.reshape(1, 1),
    )
    return out


# speedup vs baseline: 1.9678x; 1.0013x over previous
import functools

import jax
import jax.numpy as jnp
from jax import lax
from jax.experimental import pallas as pl
from jax.experimental.pallas import tpu as pltpu
from jax.experimental.pallas import tpu_sc as plsc

_N = 10000
_E = 320000
_D = 128
_H = 64

_NC = 2
_NS = 16
_NW = _NC * _NS
_EPT = _E // _NW
_CH = 128
_NFULL = _EPT // _CH
_REM = _EPT - _NFULL * _CH
_RPT = 624
_RTAIL = _N - _NS * _RPT

_MIN32 = -(2 ** 31)

_HROWS = _E // _D


_HUNROLL = 10


def _hist_body(colb_ref, out_ref):
    ioc = lax.broadcasted_iota(jnp.int32, (_D, 1), 0)

    def body(t, accs):
        a0, a1 = accs
        base = t * _HUNROLL
        for k in range(_HUNROLL):
            crow = colb_ref[pl.ds(base + k, 1), :]
            ohi = ((crow >> 7) == ioc).astype(jnp.bfloat16)
            olo = ((crow & 127) == ioc).astype(jnp.bfloat16)
            h = lax.dot_general(ohi, olo, (((1,), (1,)), ((), ())),
                                preferred_element_type=jnp.float32)
            if k % 2 == 0:
                a0 = a0 + h
            else:
                a1 = a1 + h
        return a0, a1

    z = jnp.zeros((_D, _D), jnp.float32)
    a0, a1 = lax.fori_loop(0, _HROWS // _HUNROLL, body, (z, z))
    out_ref[...] = a0 + a1


def _hist_call(col2d):
    return pl.pallas_call(
        _hist_body,
        out_shape=jax.ShapeDtypeStruct((_D, _D), jnp.float32),
    )(col2d)


def _fill_idx(dst, src, off):
    for k in range(_CH // 16):
        dst[pl.ds(k * 16, 16)] = src[pl.ds(off + k * 16, 16)]


def _segsum_kernel(y_hbm, row_hbm, col_hbm, zeros_hbm, out_hbm,
                   idxc, rowv0, rowv1, colv, rowr, colr,
                   gbuf0, gbuf1, gbufr, acc_sh, sem0, sem1):
    c = lax.axis_index("c")
    s = lax.axis_index("s")
    pltpu.sync_copy(zeros_hbm, acc_sh.at[pl.ds(s * _RPT, _RPT)])

    @pl.when(s == _NS - 1)
    def _():
        pltpu.sync_copy(zeros_hbm.at[pl.ds(0, _RTAIL)],
                        acc_sh.at[pl.ds(_NS * _RPT, _RTAIL)])

    base = (c * _NS + s) * _EPT
    pltpu.sync_copy(col_hbm.at[pl.ds(base, _NFULL * _CH)], idxc)
    plsc.subcore_barrier()

    bufs = ((rowv0, gbuf0, sem0), (rowv1, gbuf1, sem1))

    def _issue(j, which):
        rowv, gbuf, sem = bufs[which]
        pltpu.sync_copy(row_hbm.at[pl.ds(base + j * _CH, _CH)], rowv)
        pltpu.async_copy(y_hbm.at[rowv], gbuf, sem)

    def _drain(j, which, last):
        rowv, gbuf, sem = bufs[which]
        pltpu.make_async_copy(y_hbm.at[rowv], gbuf, sem).wait()
        _fill_idx(colv, idxc, j * _CH)
        pltpu.sync_copy(gbuf, acc_sh.at[colv], add=True)

        @pl.when(jnp.logical_not(last))
        def _():
            _issue(j + 2, which)

    _issue(0, 0)
    _issue(1, 1)

    def body(t, _):
        j = 2 * t
        _drain(j, 0, j + 2 >= _NFULL)
        _drain(j + 1, 1, j + 3 >= _NFULL)
        return 0

    lax.fori_loop(0, _NFULL // 2, body, 0)
    b = base + _NFULL * _CH
    pltpu.sync_copy(row_hbm.at[pl.ds(b, _REM)], rowr)
    pltpu.sync_copy(col_hbm.at[pl.ds(b, _REM)], colr)
    pltpu.async_copy(y_hbm.at[rowr], gbufr, sem0).wait()
    pltpu.sync_copy(gbufr, acc_sh.at[colr], add=True)
    plsc.subcore_barrier()
    pltpu.sync_copy(acc_sh.at[pl.ds(s * _RPT, _RPT)],
                    out_hbm.at[c, pl.ds(s * _RPT, _RPT)])

    @pl.when(s == _NS - 1)
    def _():
        pltpu.sync_copy(acc_sh.at[pl.ds(_NS * _RPT, _RTAIL)],
                        out_hbm.at[c, pl.ds(_NS * _RPT, _RTAIL)])


@functools.cache
def _sc_kernels():
    mesh = plsc.VectorSubcoreMesh(core_axis_name="c", subcore_axis_name="s",
                                  num_cores=_NC, num_subcores=_NS)
    segsum = pl.kernel(
        _segsum_kernel,
        out_type=jax.ShapeDtypeStruct((_NC, _N, _D), jnp.float32),
        mesh=mesh,
        scratch_types=[
            pltpu.VMEM((_NFULL * _CH,), jnp.int32),
            pltpu.VMEM((_CH,), jnp.int32),
            pltpu.VMEM((_CH,), jnp.int32),
            pltpu.VMEM((_CH,), jnp.int32),
            pltpu.VMEM((_REM,), jnp.int32),
            pltpu.VMEM((_REM,), jnp.int32),
            pltpu.VMEM((_CH, _D), jnp.float32),
            pltpu.VMEM((_CH, _D), jnp.float32),
            pltpu.VMEM((_REM, _D), jnp.float32),
            pltpu.VMEM_SHARED((_N, _D), jnp.float32),
            pltpu.SemaphoreType.DMA,
            pltpu.SemaphoreType.DMA,
        ],
    )
    return segsum


_MM_BLK = 2000


def _y_body(x_ref, w_ref, deg1_ref, y_ref):
    xw = jnp.dot(x_ref[...], w_ref[...], preferred_element_type=jnp.float32)
    deg = deg1_ref[...] + 1.0
    y_ref[...] = xw * lax.rsqrt(deg)


def _y_call(x, w, deg1):
    grid = (_N // _MM_BLK,)
    return pl.pallas_call(
        _y_body,
        grid=grid,
        in_specs=[
            pl.BlockSpec((_MM_BLK, _D), lambda i: (i, 0)),
            pl.BlockSpec((_D, _D), lambda i: (0, 0)),
            pl.BlockSpec((_MM_BLK, 1), lambda i: (i, 0)),
        ],
        out_specs=pl.BlockSpec((_MM_BLK, _D), lambda i: (i, 0)),
        out_shape=jax.ShapeDtypeStruct((_N, _D), jnp.float32),
    )(x, w, deg1)


def _f2u(b):
    return jnp.where(b >= 0, b ^ _MIN32, ~b)


def _u2f(u):
    b = jnp.where(u < 0, u ^ _MIN32, ~u)
    return lax.bitcast_convert_type(b, jnp.float32)


def _final_body(accp_ref, y_ref, deg1_ref, bg_ref, w1_ref, b1_ref,
                w2_ref, b2_ref, w3t_ref, b3_ref, out_ref, u_ref):
    deg = deg1_ref[...] + 1.0
    dis = lax.rsqrt(deg)
    conv = (accp_ref[0] + accp_ref[1] + y_ref[...]) * dis + bg_ref[...]
    b = lax.bitcast_convert_type(conv, jnp.int32)
    u_ref[...] = _f2u(b)

    khalf = (_N // 2) - 1

    def bit_body(i, carry):
        prefix, kk, mh = carry
        bit = lax.shift_left(jnp.int32(1), jnp.int32(31) - i)
        u = u_ref[...]
        m = (u & (mh | bit)) == prefix
        cnt0 = jnp.sum(jnp.where(m, 1.0, 0.0), axis=0, keepdims=True)
        go1 = kk >= cnt0
        prefix = jnp.where(go1, prefix | bit, prefix)
        kk = jnp.where(go1, kk - cnt0, kk)
        return prefix, kk, mh | bit

    prefix0 = jnp.zeros((1, _D), jnp.int32)
    kk0 = jnp.full((1, _D), float(khalf), jnp.float32)
    key, kkf, _ = lax.fori_loop(0, 32, bit_body,
                                (prefix0, kk0, jnp.int32(0)))

    u = u_ref[...]
    v1 = _u2f(key)
    c_eq = jnp.sum(jnp.where(u == key, 1.0, 0.0), axis=0, keepdims=True)
    below = float(khalf) - kkf
    has2 = (below + c_eq) >= float(khalf + 2)

    us = u ^ _MIN32
    keys_s = key ^ _MIN32
    cand = jnp.where(us > keys_s, us, jnp.int32(2 ** 31 - 1))
    v2 = _u2f(jnp.min(cand, axis=0, keepdims=True) ^ _MIN32)
    v2 = jnp.where(has2, v1, v2)
    med = 0.5 * (v1 + v2)

    h1 = jnp.tanh(jnp.dot(med, w1_ref[...],
                          preferred_element_type=jnp.float32) + b1_ref[...])
    h2 = jnp.tanh(jnp.dot(h1, w2_ref[...],
                          preferred_element_type=jnp.float32) + b2_ref[...])
    out_ref[...] = (jnp.sum(h2 * w3t_ref[...], axis=1, keepdims=True)
                    + b3_ref[...])


def _final_call(accp, y, deg1, bg, w1, b1, w2, b2, w3t, b3):
    return pl.pallas_call(
        _final_body,
        out_shape=jax.ShapeDtypeStruct((1, 1), jnp.float32),
        scratch_shapes=[pltpu.VMEM((_N, _D), jnp.int32)],
    )(accp, y, deg1, bg, w1, b1, w2, b2, w3t, b3)


def kernel(x, edge_index, W_gcn, b_gcn, W1, b1, W2, b2, W3, b3):
    row = edge_index[0]
    col = edge_index[1]
    zerosD = jnp.zeros((_RPT, _D), jnp.float32)

    segsum_k = _sc_kernels()
    hist = _hist_call(col.reshape(_E // _D, _D))
    deg1 = hist.reshape(-1)[:_N].reshape(_N, 1)
    y = _y_call(x, W_gcn, deg1)
    accp = segsum_k(y, row, col, zerosD)
    out = _final_call(
        accp, y, deg1,
        b_gcn.reshape(1, _D),
        W1, b1.reshape(1, _H),
        W2, b2.reshape(1, _H),
        W3.reshape(1, _H), b3.reshape(1, 1),
    )
    return out
